# baseline (device time: 21151 ns/iter reference)
import jax
import jax.numpy as jnp
from jax import lax
from jax.experimental import pallas as pl
from jax.experimental.pallas import tpu as pltpu


def kernel(A, B):
    m, k = A.shape
    _, n = B.shape

    C = 4
    mc = m // C
    assert m % C == 0

    def body(
        a_hbm,
        b_hbm,
        out_hbm,
        a_vmem,
        b_vmem,
        send_ref,
        recv_ref,
        out_vmem,
        a_sems,
        b_sem,
        out_sems,
        send_sems,
        recv_sems,
    ):
        my_x = lax.axis_index("x")
        my_y = lax.axis_index("y")
        peer = (1 - my_x, my_y)

        b_dma = pltpu.make_async_copy(b_hbm, b_vmem, b_sem)
        b_dma.start()
        a_dmas = []
        for c in range(C):
            rows = pl.ds(c * mc, mc)
            dma = pltpu.make_async_copy(
                a_hbm.at[rows, :], a_vmem.at[c], a_sems.at[c]
            )
            dma.start()
            a_dmas.append(dma)

        barrier = pltpu.get_barrier_semaphore()
        pl.semaphore_signal(
            barrier, inc=1, device_id=peer, device_id_type=pl.DeviceIdType.MESH
        )
        pl.semaphore_wait(barrier, 1)

        b_dma.wait()
        b16 = b_vmem[...].astype(jnp.bfloat16)

        rdmas = []
        for c in range(C):
            a_dmas[c].wait()
            partial = lax.dot_general(
                a_vmem[c].astype(jnp.bfloat16),
                b16,
                (((1,), (0,)), ((), ())),
                preferred_element_type=jnp.float32,
            )
            send_ref[c] = partial.astype(jnp.bfloat16)
            rdma = pltpu.make_async_remote_copy(
                src_ref=send_ref.at[c],
                dst_ref=recv_ref.at[c],
                send_sem=send_sems.at[c],
                recv_sem=recv_sems.at[c],
                device_id=peer,
                device_id_type=pl.DeviceIdType.MESH,
            )
            rdma.start()
            rdmas.append(rdma)

        out_dmas = []
        for c in range(C):
            rows = pl.ds(c * mc, mc)
            rdmas[c].wait_recv()
            out_vmem[c] = (
                send_ref[c].astype(jnp.float32) + recv_ref[c].astype(jnp.float32)
            ).astype(jnp.bfloat16)
            dma = pltpu.make_async_copy(
                out_vmem.at[c], out_hbm.at[rows, :], out_sems.at[c]
            )
            dma.start()
            out_dmas.append(dma)
        for c in range(C):
            out_dmas[c].wait()
            rdmas[c].wait_send()

    return pl.pallas_call(
        body,
        out_shape=jax.ShapeDtypeStruct((m, n), jnp.bfloat16),
        in_specs=[
            pl.BlockSpec(memory_space=pl.ANY),
            pl.BlockSpec(memory_space=pl.ANY),
        ],
        out_specs=pl.BlockSpec(memory_space=pl.ANY),
        scratch_shapes=[
            pltpu.VMEM((C, mc, k), jnp.float32),
            pltpu.VMEM((k, n), jnp.float32),
            pltpu.VMEM((C, mc, n), jnp.bfloat16),
            pltpu.VMEM((C, mc, n), jnp.bfloat16),
            pltpu.VMEM((C, mc, n), jnp.bfloat16),
            pltpu.SemaphoreType.DMA((C,)),
            pltpu.SemaphoreType.DMA,
            pltpu.SemaphoreType.DMA((C,)),
            pltpu.SemaphoreType.DMA((C,)),
            pltpu.SemaphoreType.DMA((C,)),
        ],
        compiler_params=pltpu.CompilerParams(collective_id=0),
    )(A, B)


# device time: 20318 ns/iter; 1.0410x vs baseline; 1.0410x over previous
import jax
import jax.numpy as jnp
from jax import lax
from jax.experimental import pallas as pl
from jax.experimental.pallas import tpu as pltpu


def kernel(A, B):
    m, k = A.shape
    _, n = B.shape

    sizes = [96, 224, 224, 224]
    assert sum(sizes) == m
    offs = [sum(sizes[:c]) for c in range(len(sizes))]
    C = len(sizes)

    def body(a_ref, b_ref, out_ref, send_ref, recv_ref, send_sems, recv_sems):
        my_x = lax.axis_index("x")
        my_y = lax.axis_index("y")
        peer = (1 - my_x, my_y)

        barrier = pltpu.get_barrier_semaphore()
        pl.semaphore_signal(
            barrier, inc=1, device_id=peer, device_id_type=pl.DeviceIdType.MESH
        )

        b16 = b_ref[...].astype(jnp.bfloat16)

        rdmas = []
        for c in range(C):
            rows = pl.ds(offs[c], sizes[c])
            partial = lax.dot_general(
                a_ref[rows, :].astype(jnp.bfloat16),
                b16,
                (((1,), (0,)), ((), ())),
                preferred_element_type=jnp.float32,
            )
            send_ref[rows, :] = partial.astype(jnp.bfloat16)
            if c == 0:
                pl.semaphore_wait(barrier, 1)
            rdma = pltpu.make_async_remote_copy(
                src_ref=send_ref.at[rows, :],
                dst_ref=recv_ref.at[rows, :],
                send_sem=send_sems.at[c],
                recv_sem=recv_sems.at[c],
                device_id=peer,
                device_id_type=pl.DeviceIdType.MESH,
            )
            rdma.start()
            rdmas.append(rdma)

        for c in range(C):
            rows = pl.ds(offs[c], sizes[c])
            rdmas[c].wait_recv()
            out_ref[rows, :] = (
                send_ref[rows, :].astype(jnp.float32)
                + recv_ref[rows, :].astype(jnp.float32)
            ).astype(jnp.bfloat16)
        for c in range(C):
            rdmas[c].wait_send()

    return pl.pallas_call(
        body,
        out_shape=jax.ShapeDtypeStruct((m, n), jnp.bfloat16),
        in_specs=[
            pl.BlockSpec(memory_space=pltpu.VMEM),
            pl.BlockSpec(memory_space=pltpu.VMEM),
        ],
        out_specs=pl.BlockSpec(memory_space=pltpu.VMEM),
        scratch_shapes=[
            pltpu.VMEM((m, n), jnp.bfloat16),
            pltpu.VMEM((m, n), jnp.bfloat16),
            pltpu.SemaphoreType.DMA((C,)),
            pltpu.SemaphoreType.DMA((C,)),
        ],
        compiler_params=pltpu.CompilerParams(collective_id=0),
    )(A, B)


# device time: 18422 ns/iter; 1.1481x vs baseline; 1.1029x over previous
import jax
import jax.numpy as jnp
from jax import lax
from jax.experimental import pallas as pl
from jax.experimental.pallas import tpu as pltpu


def kernel(A, B):
    m, k = A.shape
    _, n = B.shape
    mh = m // 2

    nsz = [256, 256, 128, 128]
    assert sum(nsz) == n
    noff = [sum(nsz[:c]) for c in range(len(nsz))]
    NB = len(nsz)

    def body(
        a_ref,
        b_ref,
        out_ref,
        a_send,
        b_send,
        a_other,
        b_other,
        acc_ref,
        ax_ssem,
        ax_rsem,
        ay_ssem,
        ay_rsem,
        bx_ssems,
        bx_rsems,
    ):
        my_x = lax.axis_index("x")
        my_y = lax.axis_index("y")
        xpeer = (1 - my_x, my_y)
        ypeer = (my_x, 1 - my_y)

        barrier = pltpu.get_barrier_semaphore()
        for nbr in (xpeer, ypeer):
            pl.semaphore_signal(
                barrier, inc=1, device_id=nbr, device_id_type=pl.DeviceIdType.MESH
            )

        a_send[...] = a_ref[...].astype(jnp.bfloat16)

        pl.semaphore_wait(barrier, 2)

        my_rows = pl.ds(my_y * mh, mh)
        other_rows = pl.ds((1 - my_y) * mh, mh)
        rdma_ax = pltpu.make_async_remote_copy(
            src_ref=a_send.at[my_rows, :],
            dst_ref=a_other.at[my_rows, :],
            send_sem=ax_ssem,
            recv_sem=ax_rsem,
            device_id=xpeer,
            device_id_type=pl.DeviceIdType.MESH,
        )
        rdma_ax.start()

        rdma_bx = []
        for c in range(NB):
            cols = pl.ds(noff[c], nsz[c])
            b_send[:, cols] = b_ref[:, cols].astype(jnp.bfloat16)
            rdma = pltpu.make_async_remote_copy(
                src_ref=b_send.at[:, cols],
                dst_ref=b_other.at[:, cols],
                send_sem=bx_ssems.at[c],
                recv_sem=bx_rsems.at[c],
                device_id=xpeer,
                device_id_type=pl.DeviceIdType.MESH,
            )
            rdma.start()
            rdma_bx.append(rdma)

        acc_ref[...] = lax.dot_general(
            a_send[...],
            b_send[...],
            (((1,), (0,)), ((), ())),
            preferred_element_type=jnp.float32,
        )

        rdma_ax.wait_recv()
        rdma_ay = pltpu.make_async_remote_copy(
            src_ref=a_other.at[my_rows, :],
            dst_ref=a_other.at[my_rows, :],
            send_sem=ay_ssem,
            recv_sem=ay_rsem,
            device_id=ypeer,
            device_id_type=pl.DeviceIdType.MESH,
        )
        rdma_ay.start()

        def add_chunk(rows, c):
            cols = pl.ds(noff[c], nsz[c])
            dot = lax.dot_general(
                a_other[rows, :],
                b_other[:, cols],
                (((1,), (0,)), ((), ())),
                preferred_element_type=jnp.float32,
            )
            out_ref[rows, cols] = (acc_ref[rows, cols] + dot).astype(jnp.bfloat16)

        for c in range(NB - 1):
            rdma_bx[c].wait_recv()
            add_chunk(my_rows, c)

        rdma_ay.wait_recv()
        for c in range(NB - 1):
            add_chunk(other_rows, c)

        rdma_bx[NB - 1].wait_recv()
        add_chunk(my_rows, NB - 1)
        add_chunk(other_rows, NB - 1)

        rdma_ax.wait_send()
        rdma_ay.wait_send()
        for c in range(NB):
            rdma_bx[c].wait_send()

    return pl.pallas_call(
        body,
        out_shape=jax.ShapeDtypeStruct((m, n), jnp.bfloat16),
        in_specs=[
            pl.BlockSpec(memory_space=pltpu.VMEM),
            pl.BlockSpec(memory_space=pltpu.VMEM),
        ],
        out_specs=pl.BlockSpec(memory_space=pltpu.VMEM),
        scratch_shapes=[
            pltpu.VMEM((m, k), jnp.bfloat16),
            pltpu.VMEM((k, n), jnp.bfloat16),
            pltpu.VMEM((m, k), jnp.bfloat16),
            pltpu.VMEM((k, n), jnp.bfloat16),
            pltpu.VMEM((m, n), jnp.float32),
            pltpu.SemaphoreType.DMA,
            pltpu.SemaphoreType.DMA,
            pltpu.SemaphoreType.DMA,
            pltpu.SemaphoreType.DMA,
            pltpu.SemaphoreType.DMA((NB,)),
            pltpu.SemaphoreType.DMA((NB,)),
        ],
        compiler_params=pltpu.CompilerParams(collective_id=0),
    )(A, B)


# device time: 18396 ns/iter; 1.1498x vs baseline; 1.0014x over previous
import jax
import jax.numpy as jnp
from jax import lax
from jax.experimental import pallas as pl
from jax.experimental.pallas import tpu as pltpu


def kernel(A, B):
    m, k = A.shape
    _, n = B.shape
    mh = m // 2

    nsz = [256, 256, 128, 128]
    assert sum(nsz) == n
    noff = [sum(nsz[:c]) for c in range(len(nsz))]
    NB = len(nsz)

    def body(
        a_ref,
        b_ref,
        out_ref,
        a_send,
        b_send,
        a_other,
        b_other,
        acc_ref,
        ax_ssem,
        ax_rsem,
        ay_ssem,
        ay_rsem,
        bx_ssems,
        bx_rsems,
    ):
        my_x = lax.axis_index("x")
        my_y = lax.axis_index("y")
        xpeer = (1 - my_x, my_y)
        ypeer = (my_x, 1 - my_y)

        barrier = pltpu.get_barrier_semaphore()
        for nbr in (xpeer, ypeer):
            pl.semaphore_signal(
                barrier, inc=1, device_id=nbr, device_id_type=pl.DeviceIdType.MESH
            )

        a_send[...] = a_ref[...].astype(jnp.bfloat16)
        cols0 = pl.ds(noff[0], nsz[0])
        b_send[:, cols0] = b_ref[:, cols0].astype(jnp.bfloat16)

        pl.semaphore_wait(barrier, 2)

        my_rows = pl.ds(my_y * mh, mh)
        other_rows = pl.ds((1 - my_y) * mh, mh)
        rdma_ax = pltpu.make_async_remote_copy(
            src_ref=a_send.at[my_rows, :],
            dst_ref=a_other.at[my_rows, :],
            send_sem=ax_ssem,
            recv_sem=ax_rsem,
            device_id=xpeer,
            device_id_type=pl.DeviceIdType.MESH,
        )
        rdma_ax.start()

        rdma_bx = []
        for c in range(NB):
            cols = pl.ds(noff[c], nsz[c])
            if c > 0:
                b_send[:, cols] = b_ref[:, cols].astype(jnp.bfloat16)
            rdma = pltpu.make_async_remote_copy(
                src_ref=b_send.at[:, cols],
                dst_ref=b_other.at[:, cols],
                send_sem=bx_ssems.at[c],
                recv_sem=bx_rsems.at[c],
                device_id=xpeer,
                device_id_type=pl.DeviceIdType.MESH,
            )
            rdma.start()
            rdma_bx.append(rdma)

        acc_ref[...] = lax.dot_general(
            a_send[...],
            b_send[...],
            (((1,), (0,)), ((), ())),
            preferred_element_type=jnp.float32,
        )

        rdma_ax.wait_recv()
        rdma_ay = pltpu.make_async_remote_copy(
            src_ref=a_other.at[my_rows, :],
            dst_ref=a_other.at[my_rows, :],
            send_sem=ay_ssem,
            recv_sem=ay_rsem,
            device_id=ypeer,
            device_id_type=pl.DeviceIdType.MESH,
        )
        rdma_ay.start()

        def add_chunk(rows, c):
            cols = pl.ds(noff[c], nsz[c])
            dot = lax.dot_general(
                a_other[rows, :],
                b_other[:, cols],
                (((1,), (0,)), ((), ())),
                preferred_element_type=jnp.float32,
            )
            out_ref[rows, cols] = (acc_ref[rows, cols] + dot).astype(jnp.bfloat16)

        for c in range(NB - 1):
            rdma_bx[c].wait_recv()
            add_chunk(my_rows, c)

        rdma_ay.wait_recv()
        for c in range(NB - 1):
            add_chunk(other_rows, c)

        rdma_bx[NB - 1].wait_recv()
        add_chunk(my_rows, NB - 1)
        add_chunk(other_rows, NB - 1)

        rdma_ax.wait_send()
        rdma_ay.wait_send()
        for c in range(NB):
            rdma_bx[c].wait_send()

    return pl.pallas_call(
        body,
        out_shape=jax.ShapeDtypeStruct((m, n), jnp.bfloat16),
        in_specs=[
            pl.BlockSpec(memory_space=pltpu.VMEM),
            pl.BlockSpec(memory_space=pltpu.VMEM),
        ],
        out_specs=pl.BlockSpec(memory_space=pltpu.VMEM),
        scratch_shapes=[
            pltpu.VMEM((m, k), jnp.bfloat16),
            pltpu.VMEM((k, n), jnp.bfloat16),
            pltpu.VMEM((m, k), jnp.bfloat16),
            pltpu.VMEM((k, n), jnp.bfloat16),
            pltpu.VMEM((m, n), jnp.float32),
            pltpu.SemaphoreType.DMA,
            pltpu.SemaphoreType.DMA,
            pltpu.SemaphoreType.DMA,
            pltpu.SemaphoreType.DMA,
            pltpu.SemaphoreType.DMA((NB,)),
            pltpu.SemaphoreType.DMA((NB,)),
        ],
        compiler_params=pltpu.CompilerParams(collective_id=0),
    )(A, B)
